# baseline (speedup 1.0000x reference)
import functools

import jax
import jax.numpy as jnp
from jax import lax
from jax.experimental import pallas as pl
from jax.experimental.pallas import tpu as pltpu
from jax.experimental.pallas import tpu_sc as plsc

N = 10000
E = 320000
D = 128
G = 128

_NW = 32
_CH = 64
_EROWS = 5120
_EPAD = _EROWS * _CH - E
_RPW = _EROWS // _NW
_NPAD = 10240
_RPT = _NPAD // 16


def _sc_agg_body(h_hbm, src_hbm, dst_hbm, out0_hbm, out1_hbm,
                 sidx, d0, d1, r0, r1, acc,
                 ism, g0, g1, s0, s1, q0, q1):
    c = lax.axis_index("c")
    s = lax.axis_index("s")
    wid = s * 2 + c
    ebase = wid * (_RPW * _CH)
    rows = [r0, r1]
    didx = [d0, d1]
    gsem = [g0, g1]
    ssem = [s0, s1]
    dsem = [q0, q1]

    pltpu.async_copy(src_hbm.at[pl.ds(ebase, _RPW * _CH)], sidx, ism).wait()

    def zb(r, carry):
        for c8 in range(8):
            r0[r, pl.ds(c8 * 16, 16)] = jnp.zeros((16,), jnp.float32)
        return carry
    lax.fori_loop(0, _CH, zb, None)
    for r in range(_RPT // _CH):
        pltpu.sync_copy(r0, acc.at[pl.ds(s * _RPT + r * _CH, _CH)])
    plsc.subcore_barrier()

    def issue_chunk(k, j):
        pltpu.async_copy(dst_hbm.at[pl.ds(ebase + k * _CH, _CH)],
                         didx[j], dsem[j])
        pltpu.async_copy(h_hbm.at[sidx.at[pl.ds(k * _CH, _CH)]],
                         rows[j], gsem[j])

    def issue_scatter(j):
        pltpu.async_copy(rows[j], acc.at[didx[j]], ssem[j], add=True)

    def wait_chunk(j):
        pltpu.make_async_copy(dst_hbm.at[pl.ds(0, _CH)], didx[j],
                              dsem[j]).wait()
        pltpu.make_async_copy(h_hbm.at[sidx.at[pl.ds(0, _CH)]], rows[j],
                              gsem[j]).wait()

    def wait_scatter(j):
        pltpu.make_async_copy(rows[j], acc.at[didx[j]], ssem[j]).wait()

    issue_chunk(0, 0)

    def step(i, carry):
        for j in range(2):
            k = 2 * i + j
            jn = (j + 1) % 2

            @pl.when(k >= 1)
            def _():
                wait_scatter(jn)

            @pl.when(k + 1 < _RPW)
            def _():
                issue_chunk(k + 1, jn)

            wait_chunk(j)
            issue_scatter(j)
        return carry
    lax.fori_loop(0, _RPW // 2, step, None)
    wait_scatter(1)

    plsc.subcore_barrier()

    @pl.when(c == 0)
    def _():
        pltpu.sync_copy(acc.at[pl.ds(s * _RPT, _RPT)],
                        out0_hbm.at[pl.ds(s * _RPT, _RPT)])

    @pl.when(c == 1)
    def _():
        pltpu.sync_copy(acc.at[pl.ds(s * _RPT, _RPT)],
                        out1_hbm.at[pl.ds(s * _RPT, _RPT)])


@functools.cache
def _get_sc_agg():
  return pl.kernel(
    _sc_agg_body,
    out_type=[jax.ShapeDtypeStruct((_NPAD, D), jnp.float32),
              jax.ShapeDtypeStruct((_NPAD, D), jnp.float32)],
    mesh=plsc.VectorSubcoreMesh(core_axis_name="c", subcore_axis_name="s",
                                num_cores=2, num_subcores=16),
    scratch_types=[
        pltpu.VMEM((_RPW * _CH,), jnp.int32),
        pltpu.VMEM((_CH,), jnp.int32),
        pltpu.VMEM((_CH,), jnp.int32),
        pltpu.VMEM((_CH, D), jnp.float32),
        pltpu.VMEM((_CH, D), jnp.float32),
        pltpu.VMEM_SHARED((_NPAD, D), jnp.float32),
        pltpu.SemaphoreType.DMA,
        pltpu.SemaphoreType.DMA,
        pltpu.SemaphoreType.DMA,
        pltpu.SemaphoreType.DMA,
        pltpu.SemaphoreType.DMA,
        pltpu.SemaphoreType.DMA,
        pltpu.SemaphoreType.DMA,
    ],
  )

_BN = 1000
_NB = N // _BN
_PREC = lax.Precision.DEFAULT


def _mlp_body(h_ref, a0_ref, a1_ref, w1_ref, b1_ref, w2_ref, b2_ref,
              h2_ref, sum_ref, sq_ref):
    i = pl.program_id(0)
    hin = h_ref[...] + a0_ref[...] + a1_ref[...]
    z = jnp.dot(hin, w1_ref[...], precision=_PREC,
                preferred_element_type=jnp.float32) + b1_ref[...]
    z = jnp.maximum(z, 0.0)
    h2 = jnp.dot(z, w2_ref[...], precision=_PREC,
                 preferred_element_type=jnp.float32) + b2_ref[...]
    h2 = jnp.maximum(h2, 0.0)
    h2_ref[...] = h2
    ps = jnp.sum(h2, axis=0, keepdims=True)
    pq = jnp.sum(h2 * h2, axis=0, keepdims=True)

    @pl.when(i == 0)
    def _():
        sum_ref[...] = ps
        sq_ref[...] = pq

    @pl.when(i != 0)
    def _():
        sum_ref[...] += ps
        sq_ref[...] += pq


_row_spec = pl.BlockSpec((_BN, D), lambda i: (i, 0))
_w_spec = pl.BlockSpec((D, D), lambda i: (0, 0))
_v_spec = pl.BlockSpec((1, D), lambda i: (0, 0))

_mlp = pl.pallas_call(
    _mlp_body,
    grid=(_NB,),
    in_specs=[
        _row_spec,
        pl.BlockSpec((_BN, D), lambda i: (i, 0)),
        pl.BlockSpec((_BN, D), lambda i: (i, 0)),
        _w_spec, _v_spec, _w_spec, _v_spec,
    ],
    out_specs=[_row_spec, _v_spec, _v_spec],
    out_shape=[
        jax.ShapeDtypeStruct((N, D), jnp.float32),
        jax.ShapeDtypeStruct((1, D), jnp.float32),
        jax.ShapeDtypeStruct((1, D), jnp.float32),
    ],
)


def _norm_body(h2_ref, sum_ref, sq_ref, g_ref, b_ref, out_ref):
    mean = sum_ref[...] * (1.0 / N)
    var = sq_ref[...] * (1.0 / N) - mean * mean
    inv = lax.rsqrt(var + 1e-5) * g_ref[...]
    out_ref[...] = (h2_ref[...] - mean) * inv + b_ref[...]


_norm = pl.pallas_call(
    _norm_body,
    grid=(_NB,),
    in_specs=[_row_spec, _v_spec, _v_spec, _v_spec, _v_spec],
    out_specs=_row_spec,
    out_shape=jax.ShapeDtypeStruct((N, D), jnp.float32),
)


def _pool_body(h2_ref, sum_ref, sq_ref, g_ref, b_ref, batch_ref, out_ref,
               acc, cnt):
    i = pl.program_id(0)

    @pl.when(i == 0)
    def _():
        acc[...] = jnp.zeros_like(acc)
        cnt[...] = jnp.zeros_like(cnt)

    mean = sum_ref[...] * (1.0 / N)
    var = sq_ref[...] * (1.0 / N) - mean * mean
    inv = lax.rsqrt(var + 1e-5) * g_ref[...]
    y = (h2_ref[...] - mean) * inv + b_ref[...]

    bvec = batch_ref[0, 0, :]
    oh = (bvec[:, None] == lax.broadcasted_iota(jnp.int32, (_BN, G), 1)
          ).astype(jnp.float32)
    acc[...] += lax.dot_general(oh, y, (((0,), (0,)), ((), ())),
                                precision=_PREC,
                                preferred_element_type=jnp.float32)
    cnt[...] += lax.dot_general(oh, jnp.ones((_BN, 8), jnp.float32),
                                (((0,), (0,)), ((), ())),
                                precision=_PREC,
                                preferred_element_type=jnp.float32)

    @pl.when(i == _NB - 1)
    def _():
        out_ref[...] = acc[...] / jnp.maximum(cnt[...][:, :1], 1.0)


_pool = pl.pallas_call(
    _pool_body,
    grid=(_NB,),
    in_specs=[
        _row_spec, _v_spec, _v_spec, _v_spec, _v_spec,
        pl.BlockSpec((1, 1, _BN), lambda i: (i, 0, 0)),
    ],
    out_specs=pl.BlockSpec((G, G), lambda i: (0, 0)),
    out_shape=jax.ShapeDtypeStruct((G, G), jnp.float32),
    scratch_shapes=[
        pltpu.VMEM((G, G), jnp.float32),
        pltpu.VMEM((G, 8), jnp.float32),
    ],
)


@jax.jit
def kernel(x, edge_index, batch, params):
    pad_src = jnp.zeros((_EPAD,), jnp.int32)
    pad_dst = N + (jnp.arange(_EPAD, dtype=jnp.int32) % (_NPAD - N))
    src = jnp.concatenate([edge_index[0], pad_src])
    dst = jnp.concatenate([edge_index[1], pad_dst])
    batch3 = batch.reshape(_NB, 1, _BN)
    h = x
    for l, (W1, b1, W2, b2, gamma, beta) in enumerate(params):
        agg_a, agg_b = _get_sc_agg()(h, src, dst)
        h2, ssum, ssq = _mlp(h, agg_a, agg_b, W1, b1.reshape(1, D),
                             W2, b2.reshape(1, D))
        if l < len(params) - 1:
            h = _norm(h2, ssum, ssq, gamma.reshape(1, D), beta.reshape(1, D))
        else:
            return _pool(h2, ssum, ssq, gamma.reshape(1, D),
                         beta.reshape(1, D), batch3)


# speedup vs baseline: 3.8688x; 3.8688x over previous
import functools

import jax
import jax.numpy as jnp
from jax import lax
from jax.experimental import pallas as pl
from jax.experimental.pallas import tpu as pltpu
from jax.experimental.pallas import tpu_sc as plsc

N = 10000
E = 320000
D = 128
G = 128

_NW = 32
_CH = 128
_NCHUNK = E // _CH
_MAXC = 80
_NPAD = 10240
_RPT = _NPAD // 16


def _sc_agg_body(h_hbm, src_hbm, dst_hbm, out0_hbm, out1_hbm,
                 sidx, d0, d1, r0, r1, acc,
                 ism, g0, g1, s0, s1, q0, q1):
    c = lax.axis_index("c")
    s = lax.axis_index("s")
    wid = s * 2 + c
    nw = jnp.where(wid < 2, _MAXC, 78)
    ebase = (78 * wid + 2 * jnp.minimum(wid, 2)) * _CH
    rows = [r0, r1]
    didx = [d0, d1]
    gsem = [g0, g1]
    ssem = [s0, s1]
    dsem = [q0, q1]

    pltpu.async_copy(src_hbm.at[pl.ds(ebase, 78 * _CH)],
                     sidx.at[pl.ds(0, 78 * _CH)], ism).wait()

    @pl.when(wid < 2)
    def _():
        pltpu.async_copy(src_hbm.at[pl.ds(ebase + 78 * _CH, 2 * _CH)],
                         sidx.at[pl.ds(78 * _CH, 2 * _CH)], ism).wait()

    def zb(r, carry):
        for c8 in range(8):
            r0[r, pl.ds(c8 * 16, 16)] = jnp.zeros((16,), jnp.float32)
        return carry
    lax.fori_loop(0, _CH, zb, None)
    for r in range(_RPT // _CH):
        pltpu.sync_copy(r0, acc.at[pl.ds(s * _RPT + r * _CH, _CH)])
    plsc.subcore_barrier()

    def issue_chunk(k, j):
        pltpu.async_copy(dst_hbm.at[pl.ds(ebase + k * _CH, _CH)],
                         didx[j], dsem[j])
        pltpu.async_copy(h_hbm.at[sidx.at[pl.ds(k * _CH, _CH)]],
                         rows[j], gsem[j])

    def issue_scatter(j):
        pltpu.async_copy(rows[j], acc.at[didx[j]], ssem[j], add=True)

    def wait_chunk(j):
        pltpu.make_async_copy(dst_hbm.at[pl.ds(0, _CH)], didx[j],
                              dsem[j]).wait()
        pltpu.make_async_copy(h_hbm.at[sidx.at[pl.ds(0, _CH)]], rows[j],
                              gsem[j]).wait()

    def wait_scatter(j):
        pltpu.make_async_copy(rows[j], acc.at[didx[j]], ssem[j]).wait()

    issue_chunk(0, 0)

    def step(i, carry):
        for j in range(2):
            k = 2 * i + j
            jn = (j + 1) % 2

            @pl.when(k >= 1)
            def _():
                wait_scatter(jn)

            @pl.when(k + 1 < nw)
            def _():
                issue_chunk(k + 1, jn)

            wait_chunk(j)
            issue_scatter(j)
        return carry
    lax.fori_loop(0, nw // 2, step, None)
    wait_scatter(1)

    plsc.subcore_barrier()

    @pl.when(c == 0)
    def _():
        pltpu.sync_copy(acc.at[pl.ds(s * _RPT, _RPT)],
                        out0_hbm.at[pl.ds(s * _RPT, _RPT)])

    @pl.when(c == 1)
    def _():
        pltpu.sync_copy(acc.at[pl.ds(s * _RPT, _RPT)],
                        out1_hbm.at[pl.ds(s * _RPT, _RPT)])


@functools.cache
def _get_sc_agg():
  return pl.kernel(
    _sc_agg_body,
    out_type=[jax.ShapeDtypeStruct((_NPAD, D), jnp.float32),
              jax.ShapeDtypeStruct((_NPAD, D), jnp.float32)],
    mesh=plsc.VectorSubcoreMesh(core_axis_name="c", subcore_axis_name="s",
                                num_cores=2, num_subcores=16),
    scratch_types=[
        pltpu.VMEM((_MAXC * _CH,), jnp.int32),
        pltpu.VMEM((_CH,), jnp.int32),
        pltpu.VMEM((_CH,), jnp.int32),
        pltpu.VMEM((_CH, D), jnp.float32),
        pltpu.VMEM((_CH, D), jnp.float32),
        pltpu.VMEM_SHARED((_NPAD, D), jnp.float32),
        pltpu.SemaphoreType.DMA,
        pltpu.SemaphoreType.DMA,
        pltpu.SemaphoreType.DMA,
        pltpu.SemaphoreType.DMA,
        pltpu.SemaphoreType.DMA,
        pltpu.SemaphoreType.DMA,
        pltpu.SemaphoreType.DMA,
    ],
  )

_BN = 1000
_NB = N // _BN
_PREC = lax.Precision.DEFAULT


def _mlp_phase0(h_ref, a0_ref, a1_ref, w1_ref, b1_ref, w2_ref, b2_ref,
                h2buf, sum_ref, sq_ref, j):
    hin = h_ref[...] + a0_ref[...] + a1_ref[...]
    z = jnp.dot(hin, w1_ref[...], precision=_PREC,
                preferred_element_type=jnp.float32) + b1_ref[...]
    z = jnp.maximum(z, 0.0)
    h2 = jnp.dot(z, w2_ref[...], precision=_PREC,
                 preferred_element_type=jnp.float32) + b2_ref[...]
    h2 = jnp.maximum(h2, 0.0)
    h2buf[pl.ds(j * _BN, _BN), :] = h2
    ps = jnp.sum(h2, axis=0, keepdims=True)
    pq = jnp.sum(h2 * h2, axis=0, keepdims=True)

    @pl.when(j == 0)
    def _():
        sum_ref[...] = ps
        sq_ref[...] = pq

    @pl.when(j != 0)
    def _():
        sum_ref[...] += ps
        sq_ref[...] += pq


def _bn_scale(sum_ref, sq_ref, g_ref):
    mean = sum_ref[...] * (1.0 / N)
    var = sq_ref[...] * (1.0 / N) - mean * mean
    inv = lax.rsqrt(var + 1e-5) * g_ref[...]
    return mean, inv


def _mlpnorm_body(h_ref, a0_ref, a1_ref, w1_ref, b1_ref, w2_ref, b2_ref,
                  g_ref, be_ref, out_ref, h2buf, sum_ref, sq_ref):
    p = pl.program_id(0)
    j = pl.program_id(1)

    @pl.when(p == 0)
    def _():
        _mlp_phase0(h_ref, a0_ref, a1_ref, w1_ref, b1_ref, w2_ref, b2_ref,
                    h2buf, sum_ref, sq_ref, j)

    @pl.when(p == 1)
    def _():
        mean, inv = _bn_scale(sum_ref, sq_ref, g_ref)
        out_ref[...] = (h2buf[pl.ds(j * _BN, _BN), :] - mean) * inv \
            + be_ref[...]


_row_p0 = pl.BlockSpec((_BN, D), lambda p, j: (j * (1 - p) + (_NB - 1) * p, 0))
_w_spec = pl.BlockSpec((D, D), lambda p, j: (0, 0))
_v_spec = pl.BlockSpec((1, D), lambda p, j: (0, 0))
_row_p1 = pl.BlockSpec((_BN, D), lambda p, j: (j * p, 0))

_mlpnorm = pl.pallas_call(
    _mlpnorm_body,
    grid=(2, _NB),
    in_specs=[_row_p0, _row_p0, _row_p0, _w_spec, _v_spec, _w_spec, _v_spec,
              _v_spec, _v_spec],
    out_specs=_row_p1,
    out_shape=jax.ShapeDtypeStruct((N, D), jnp.float32),
    scratch_shapes=[
        pltpu.VMEM((N, D), jnp.float32),
        pltpu.VMEM((1, D), jnp.float32),
        pltpu.VMEM((1, D), jnp.float32),
    ],
)


def _mlppool_body(h_ref, a0_ref, a1_ref, w1_ref, b1_ref, w2_ref, b2_ref,
                  g_ref, be_ref, batch_ref, out_ref,
                  h2buf, sum_ref, sq_ref, acc, cnt):
    p = pl.program_id(0)
    j = pl.program_id(1)

    @pl.when(p == 0)
    def _():
        _mlp_phase0(h_ref, a0_ref, a1_ref, w1_ref, b1_ref, w2_ref, b2_ref,
                    h2buf, sum_ref, sq_ref, j)

    @pl.when(p == 1)
    def _():
        mean, inv = _bn_scale(sum_ref, sq_ref, g_ref)
        y = (h2buf[pl.ds(j * _BN, _BN), :] - mean) * inv + be_ref[...]

        @pl.when(j == 0)
        def _():
            acc[...] = jnp.zeros_like(acc)
            cnt[...] = jnp.zeros_like(cnt)

        bvec = batch_ref[0, 0, :]
        oh = (bvec[:, None] == lax.broadcasted_iota(jnp.int32, (_BN, G), 1)
              ).astype(jnp.float32)
        acc[...] += lax.dot_general(oh, y, (((0,), (0,)), ((), ())),
                                    precision=_PREC,
                                    preferred_element_type=jnp.float32)
        cnt[...] += lax.dot_general(oh, jnp.ones((_BN, 8), jnp.float32),
                                    (((0,), (0,)), ((), ())),
                                    precision=_PREC,
                                    preferred_element_type=jnp.float32)

        @pl.when(j == _NB - 1)
        def _():
            out_ref[...] = acc[...] / jnp.maximum(cnt[...][:, :1], 1.0)


_mlppool = pl.pallas_call(
    _mlppool_body,
    grid=(2, _NB),
    in_specs=[_row_p0, _row_p0, _row_p0, _w_spec, _v_spec, _w_spec, _v_spec,
              _v_spec, _v_spec,
              pl.BlockSpec((1, 1, _BN), lambda p, j: (j * p, 0, 0))],
    out_specs=pl.BlockSpec((G, G), lambda p, j: (0, 0)),
    out_shape=jax.ShapeDtypeStruct((G, G), jnp.float32),
    scratch_shapes=[
        pltpu.VMEM((N, D), jnp.float32),
        pltpu.VMEM((1, D), jnp.float32),
        pltpu.VMEM((1, D), jnp.float32),
        pltpu.VMEM((G, G), jnp.float32),
        pltpu.VMEM((G, 8), jnp.float32),
    ],
)


@jax.jit
def kernel(x, edge_index, batch, params):
    src = edge_index[0]
    dst = edge_index[1]
    batch3 = batch.reshape(_NB, 1, _BN)
    h = x
    for l, (W1, b1, W2, b2, gamma, beta) in enumerate(params):
        agg_a, agg_b = _get_sc_agg()(h, src, dst)
        args = (h, agg_a, agg_b, W1, b1.reshape(1, D), W2, b2.reshape(1, D),
                gamma.reshape(1, D), beta.reshape(1, D))
        if l < len(params) - 1:
            h = _mlpnorm(*args)
        else:
            return _mlppool(*args, batch3)


# speedup vs baseline: 4.0150x; 1.0378x over previous
import functools

import jax
import jax.numpy as jnp
from jax import lax
from jax.experimental import pallas as pl
from jax.experimental.pallas import tpu as pltpu
from jax.experimental.pallas import tpu_sc as plsc

N = 10000
E = 320000
D = 128
G = 128

_NW = 32
_CH = 128
_NCHUNK = E // _CH
_MAXC = 80
_NPAD = 10240
_RPT = _NPAD // 16


def _sc_agg_body(h_hbm, src_hbm, dst_hbm, out0_hbm, out1_hbm,
                 sidx, d0, d1, r0, r1, acc,
                 ism, g0, g1, s0, s1, q0, q1):
    c = lax.axis_index("c")
    s = lax.axis_index("s")
    wid = s * 2 + c
    nw = jnp.where(wid < 2, _MAXC, 78)
    ebase = (78 * wid + 2 * jnp.minimum(wid, 2)) * _CH
    rows = [r0, r1]
    didx = [d0, d1]
    gsem = [g0, g1]
    ssem = [s0, s1]
    dsem = [q0, q1]

    pltpu.async_copy(src_hbm.at[pl.ds(ebase, 78 * _CH)],
                     sidx.at[pl.ds(0, 78 * _CH)], ism).wait()

    @pl.when(wid < 2)
    def _():
        pltpu.async_copy(src_hbm.at[pl.ds(ebase + 78 * _CH, 2 * _CH)],
                         sidx.at[pl.ds(78 * _CH, 2 * _CH)], ism).wait()

    def zb(r, carry):
        for c8 in range(8):
            r0[r, pl.ds(c8 * 16, 16)] = jnp.zeros((16,), jnp.float32)
        return carry
    lax.fori_loop(0, _CH, zb, None)
    for r in range(_RPT // _CH):
        pltpu.sync_copy(r0, acc.at[pl.ds(s * _RPT + r * _CH, _CH)])
    plsc.subcore_barrier()

    def issue_chunk(k, j):
        pltpu.async_copy(dst_hbm.at[pl.ds(ebase + k * _CH, _CH)],
                         didx[j], dsem[j])
        pltpu.async_copy(h_hbm.at[sidx.at[pl.ds(k * _CH, _CH)]],
                         rows[j], gsem[j])

    def issue_scatter(j):
        pltpu.async_copy(rows[j], acc.at[didx[j]], ssem[j], add=True)

    def wait_chunk(j):
        pltpu.make_async_copy(dst_hbm.at[pl.ds(0, _CH)], didx[j],
                              dsem[j]).wait()
        pltpu.make_async_copy(h_hbm.at[sidx.at[pl.ds(0, _CH)]], rows[j],
                              gsem[j]).wait()

    def wait_scatter(j):
        pltpu.make_async_copy(rows[j], acc.at[didx[j]], ssem[j]).wait()

    issue_chunk(0, 0)

    def step(i, carry):
        for j in range(2):
            k = 2 * i + j
            jn = (j + 1) % 2

            @pl.when(k >= 1)
            def _():
                wait_scatter(jn)

            @pl.when(k + 1 < nw)
            def _():
                issue_chunk(k + 1, jn)

            wait_chunk(j)
            issue_scatter(j)
        return carry
    lax.fori_loop(0, nw // 2, step, None)
    wait_scatter(1)

    plsc.subcore_barrier()

    @pl.when(c == 0)
    def _():
        pltpu.sync_copy(acc.at[pl.ds(s * _RPT, _RPT)],
                        out0_hbm.at[pl.ds(s * _RPT, _RPT)])

    @pl.when(c == 1)
    def _():
        pltpu.sync_copy(acc.at[pl.ds(s * _RPT, _RPT)],
                        out1_hbm.at[pl.ds(s * _RPT, _RPT)])


@functools.cache
def _get_sc_agg():
  return pl.kernel(
    _sc_agg_body,
    out_type=[jax.ShapeDtypeStruct((_NPAD, D), jnp.float32),
              jax.ShapeDtypeStruct((_NPAD, D), jnp.float32)],
    mesh=plsc.VectorSubcoreMesh(core_axis_name="c", subcore_axis_name="s",
                                num_cores=2, num_subcores=16),
    scratch_types=[
        pltpu.VMEM((_MAXC * _CH,), jnp.int32),
        pltpu.VMEM((_CH,), jnp.int32),
        pltpu.VMEM((_CH,), jnp.int32),
        pltpu.VMEM((_CH, D), jnp.float32),
        pltpu.VMEM((_CH, D), jnp.float32),
        pltpu.VMEM_SHARED((_NPAD, D), jnp.float32),
        pltpu.SemaphoreType.DMA,
        pltpu.SemaphoreType.DMA,
        pltpu.SemaphoreType.DMA,
        pltpu.SemaphoreType.DMA,
        pltpu.SemaphoreType.DMA,
        pltpu.SemaphoreType.DMA,
        pltpu.SemaphoreType.DMA,
    ],
  )

_BN = 2000
_NB = N // _BN
_PREC = lax.Precision.DEFAULT


def _mlp_phase0(h_ref, a0_ref, a1_ref, w1_ref, b1_ref, w2_ref, b2_ref,
                h2buf, sum_ref, sq_ref, j):
    hin = h_ref[...] + a0_ref[...] + a1_ref[...]
    z = jnp.dot(hin, w1_ref[...], precision=_PREC,
                preferred_element_type=jnp.float32) + b1_ref[...]
    z = jnp.maximum(z, 0.0)
    h2 = jnp.dot(z, w2_ref[...], precision=_PREC,
                 preferred_element_type=jnp.float32) + b2_ref[...]
    h2 = jnp.maximum(h2, 0.0)
    h2buf[pl.ds(j * _BN, _BN), :] = h2
    ps = jnp.sum(h2, axis=0, keepdims=True)
    pq = jnp.sum(h2 * h2, axis=0, keepdims=True)

    @pl.when(j == 0)
    def _():
        sum_ref[...] = ps
        sq_ref[...] = pq

    @pl.when(j != 0)
    def _():
        sum_ref[...] += ps
        sq_ref[...] += pq


def _bn_scale(sum_ref, sq_ref, g_ref):
    mean = sum_ref[...] * (1.0 / N)
    var = sq_ref[...] * (1.0 / N) - mean * mean
    inv = lax.rsqrt(var + 1e-5) * g_ref[...]
    return mean, inv


def _mlpnorm_body(h_ref, a0_ref, a1_ref, w1_ref, b1_ref, w2_ref, b2_ref,
                  g_ref, be_ref, out_ref, h2buf, sum_ref, sq_ref):
    p = pl.program_id(0)
    j = pl.program_id(1)

    @pl.when(p == 0)
    def _():
        _mlp_phase0(h_ref, a0_ref, a1_ref, w1_ref, b1_ref, w2_ref, b2_ref,
                    h2buf, sum_ref, sq_ref, j)

    @pl.when(p == 1)
    def _():
        mean, inv = _bn_scale(sum_ref, sq_ref, g_ref)
        out_ref[...] = (h2buf[pl.ds(j * _BN, _BN), :] - mean) * inv \
            + be_ref[...]


_row_p0 = pl.BlockSpec((_BN, D), lambda p, j: (j * (1 - p) + (_NB - 1) * p, 0))
_w_spec = pl.BlockSpec((D, D), lambda p, j: (0, 0))
_v_spec = pl.BlockSpec((1, D), lambda p, j: (0, 0))
_row_p1 = pl.BlockSpec((_BN, D), lambda p, j: (j * p, 0))

_mlpnorm = pl.pallas_call(
    _mlpnorm_body,
    grid=(2, _NB),
    in_specs=[_row_p0, _row_p0, _row_p0, _w_spec, _v_spec, _w_spec, _v_spec,
              _v_spec, _v_spec],
    out_specs=_row_p1,
    out_shape=jax.ShapeDtypeStruct((N, D), jnp.float32),
    scratch_shapes=[
        pltpu.VMEM((N, D), jnp.float32),
        pltpu.VMEM((1, D), jnp.float32),
        pltpu.VMEM((1, D), jnp.float32),
    ],
)


def _mlppool_body(h_ref, a0_ref, a1_ref, w1_ref, b1_ref, w2_ref, b2_ref,
                  g_ref, be_ref, batch_ref, out_ref,
                  h2buf, sum_ref, sq_ref, acc, cnt):
    p = pl.program_id(0)
    j = pl.program_id(1)

    @pl.when(p == 0)
    def _():
        _mlp_phase0(h_ref, a0_ref, a1_ref, w1_ref, b1_ref, w2_ref, b2_ref,
                    h2buf, sum_ref, sq_ref, j)

    @pl.when(p == 1)
    def _():
        mean, inv = _bn_scale(sum_ref, sq_ref, g_ref)
        y = (h2buf[pl.ds(j * _BN, _BN), :] - mean) * inv + be_ref[...]

        @pl.when(j == 0)
        def _():
            acc[...] = jnp.zeros_like(acc)
            cnt[...] = jnp.zeros_like(cnt)

        bvec = batch_ref[0, 0, :]
        oh = (bvec[:, None] == lax.broadcasted_iota(jnp.int32, (_BN, G), 1)
              ).astype(jnp.float32)
        acc[...] += lax.dot_general(oh, y, (((0,), (0,)), ((), ())),
                                    precision=_PREC,
                                    preferred_element_type=jnp.float32)
        cnt[...] += lax.dot_general(oh, jnp.ones((_BN, 8), jnp.float32),
                                    (((0,), (0,)), ((), ())),
                                    precision=_PREC,
                                    preferred_element_type=jnp.float32)

        @pl.when(j == _NB - 1)
        def _():
            out_ref[...] = acc[...] / jnp.maximum(cnt[...][:, :1], 1.0)


_mlppool = pl.pallas_call(
    _mlppool_body,
    grid=(2, _NB),
    in_specs=[_row_p0, _row_p0, _row_p0, _w_spec, _v_spec, _w_spec, _v_spec,
              _v_spec, _v_spec,
              pl.BlockSpec((1, 1, _BN), lambda p, j: (j * p, 0, 0))],
    out_specs=pl.BlockSpec((G, G), lambda p, j: (0, 0)),
    out_shape=jax.ShapeDtypeStruct((G, G), jnp.float32),
    scratch_shapes=[
        pltpu.VMEM((N, D), jnp.float32),
        pltpu.VMEM((1, D), jnp.float32),
        pltpu.VMEM((1, D), jnp.float32),
        pltpu.VMEM((G, G), jnp.float32),
        pltpu.VMEM((G, 8), jnp.float32),
    ],
)


@jax.jit
def kernel(x, edge_index, batch, params):
    src = edge_index[0]
    dst = edge_index[1]
    batch3 = batch.reshape(_NB, 1, _BN)
    h = x
    for l, (W1, b1, W2, b2, gamma, beta) in enumerate(params):
        agg_a, agg_b = _get_sc_agg()(h, src, dst)
        args = (h, agg_a, agg_b, W1, b1.reshape(1, D), W2, b2.reshape(1, D),
                gamma.reshape(1, D), beta.reshape(1, D))
        if l < len(params) - 1:
            h = _mlpnorm(*args)
        else:
            return _mlppool(*args, batch3)
